# baseline (speedup 1.0000x reference)
import jax
import jax.numpy as jnp
from jax.experimental import pallas as pl
from jax.experimental.pallas import tpu as pltpu

_TH = 200
_TM = 2 * _TH


def _body(x_ref, adj_a, adj_b, w1_ref, b1_ref, w2_ref, b2_ref, out_ref,
          s_ref, g_ref):
    i = pl.program_id(0)
    nblk = pl.num_programs(0) // 3

    @pl.when(i < nblk)
    def _phase0():
        s_ref[pl.ds(i * _TM, _TM), :] = jnp.dot(
            x_ref[...], w1_ref[...], preferred_element_type=jnp.float32)

    @pl.when((i >= nblk) & (i < 2 * nblk))
    def _phase1():
        j = i - nblk
        for half, aref in enumerate((adj_a, adj_b)):
            h = jnp.dot(aref[...], s_ref[...],
                        preferred_element_type=jnp.float32)
            h = jnp.maximum(h + b1_ref[...], 0.0)
            g_ref[pl.ds(j * _TM + half * _TH, _TH), :] = jnp.dot(
                h, w2_ref[...], preferred_element_type=jnp.float32)

    @pl.when(i >= 2 * nblk)
    def _phase2():
        for half, aref in enumerate((adj_a, adj_b)):
            v = jnp.dot(aref[...], g_ref[...],
                        preferred_element_type=jnp.float32)
            v = v + b2_ref[...]
            m = jnp.max(v, axis=1, keepdims=True)
            lse = jnp.log(jnp.sum(jnp.exp(v - m), axis=1, keepdims=True)) + m
            out_ref[pl.ds(half * _TH, _TH), :] = v - lse


def kernel(x, adj, W1, b1, W2, b2):
    n, nfeat = x.shape
    nhid = W1.shape[1]
    nclass = W2.shape[1]
    b1r = b1.reshape(1, nhid)
    b2r = b2.reshape(1, nclass)

    nblk = n // _TM

    def x_map(i):
        return (jnp.minimum(i, nblk - 1), 0)

    def adj_map_a(i):
        j = jnp.where(i < nblk, 0, jnp.where(i < 2 * nblk, i - nblk, i - 2 * nblk))
        return (2 * j, 0)

    def adj_map_b(i):
        j = jnp.where(i < nblk, 0, jnp.where(i < 2 * nblk, i - nblk, i - 2 * nblk))
        return (2 * j + 1, 0)

    def out_map(i):
        return (jnp.maximum(i - 2 * nblk, 0), 0)

    const = lambda i: (0, 0)

    out = pl.pallas_call(
        _body,
        grid=(3 * nblk,),
        in_specs=[
            pl.BlockSpec((_TM, nfeat), x_map),
            pl.BlockSpec((_TH, n), adj_map_a),
            pl.BlockSpec((_TH, n), adj_map_b),
            pl.BlockSpec((nfeat, nhid), const),
            pl.BlockSpec((1, nhid), const),
            pl.BlockSpec((nhid, nclass), const),
            pl.BlockSpec((1, nclass), const),
        ],
        out_specs=pl.BlockSpec((_TM, nclass), out_map),
        out_shape=jax.ShapeDtypeStruct((n, nclass), jnp.float32),
        scratch_shapes=[
            pltpu.VMEM((n, nhid), jnp.float32),
            pltpu.VMEM((n, nclass), jnp.float32),
        ],
        compiler_params=pltpu.CompilerParams(
            dimension_semantics=("arbitrary",)),
    )(x, adj, adj, W1, b1r, W2, b2r)

    return out


# speedup vs baseline: 1.0788x; 1.0788x over previous
import jax
import jax.numpy as jnp
from jax.experimental import pallas as pl
from jax.experimental.pallas import tpu as pltpu

_TM = 400
_TX = 2000


def _make_body(nx, nblk):
  def _body(x_ref, adj_ref, w1_ref, b1_ref, w2_ref, b2_ref, out_ref, s_ref, g_ref):
    i = pl.program_id(0)

    @pl.when(i < nx)
    def _phase0():
        s_ref[pl.ds(i * _TX, _TX), :] = jnp.dot(
            x_ref[...], w1_ref[...], preferred_element_type=jnp.float32)

    @pl.when((i >= nx) & (i < nx + nblk))
    def _phase1():
        h = jnp.dot(adj_ref[...], s_ref[...], preferred_element_type=jnp.float32)
        h = jnp.maximum(h + b1_ref[...], 0.0)
        g_ref[pl.ds((i - nx) * _TM, _TM), :] = jnp.dot(
            h, w2_ref[...], preferred_element_type=jnp.float32)

    @pl.when(i >= nx + nblk)
    def _phase2():
        v = jnp.dot(adj_ref[...], g_ref[...], preferred_element_type=jnp.float32)
        v = v + b2_ref[...]
        m = jnp.max(v, axis=1, keepdims=True)
        lse = jnp.log(jnp.sum(jnp.exp(v - m), axis=1, keepdims=True)) + m
        out_ref[...] = v - lse
  return _body


def kernel(x, adj, W1, b1, W2, b2):
    n, nfeat = x.shape
    nhid = W1.shape[1]
    nclass = W2.shape[1]
    b1r = b1.reshape(1, nhid)
    b2r = b2.reshape(1, nclass)

    nx = n // _TX
    nblk = n // _TM

    def x_map(i):
        return (jnp.minimum(i, nx - 1), 0)

    def adj_map(i):
        j = jnp.where(i < nx + nblk, jnp.maximum(i - nx, 0), i - nx - nblk)
        return (j, 0)

    def out_map(i):
        return (jnp.maximum(i - nx - nblk, 0), 0)

    const = lambda i: (0, 0)

    out = pl.pallas_call(
        _make_body(nx, nblk),
        grid=(nx + 2 * nblk,),
        in_specs=[
            pl.BlockSpec((_TX, nfeat), x_map),
            pl.BlockSpec((_TM, n), adj_map),
            pl.BlockSpec((nfeat, nhid), const),
            pl.BlockSpec((1, nhid), const),
            pl.BlockSpec((nhid, nclass), const),
            pl.BlockSpec((1, nclass), const),
        ],
        out_specs=pl.BlockSpec((_TM, nclass), out_map),
        out_shape=jax.ShapeDtypeStruct((n, nclass), jnp.float32),
        scratch_shapes=[
            pltpu.VMEM((n, nhid), jnp.float32),
            pltpu.VMEM((n, nclass), jnp.float32),
        ],
        compiler_params=pltpu.CompilerParams(
            dimension_semantics=("arbitrary",)),
    )(x, adj, W1, b1r, W2, b2r)

    return out


# speedup vs baseline: 1.0854x; 1.0061x over previous
import jax
import jax.numpy as jnp
from jax.experimental import pallas as pl
from jax.experimental.pallas import tpu as pltpu

_TR = 200
_B = 4


def _make_body(n, nfeat, nhid, nclass):
    nt = n // _TR
    total = 2 * nt

    def _adj_copy(adj_ref, ring_ref, sem_ref, t, slot):
        row = jax.lax.rem(t, nt) * _TR
        return pltpu.make_async_copy(
            adj_ref.at[pl.ds(row, _TR), :], ring_ref.at[slot], sem_ref.at[slot])

    def _out_copy(stage_ref, out_ref, osem_ref, t, slot):
        row = (t - nt) * _TR
        return pltpu.make_async_copy(
            stage_ref.at[slot], out_ref.at[pl.ds(row, _TR), :],
            osem_ref.at[slot])

    def _body(x_ref, adj_ref, w1_ref, b1_ref, w2_ref, b2_ref, out_ref,
              ring_ref, xbuf_ref, stage_ref, s_ref, g_ref,
              sem_ref, xsem_ref, osem_ref):
        xcopy = pltpu.make_async_copy(x_ref, xbuf_ref, xsem_ref)
        xcopy.start()
        for j in range(_B):
            _adj_copy(adj_ref, ring_ref, sem_ref, j, j).start()

        xcopy.wait()
        s_ref[...] = jnp.dot(xbuf_ref[...], w1_ref[...],
                             preferred_element_type=jnp.float32)

        def outer(step, _):
            for j in range(_B):
                t = step * _B + j
                row = jax.lax.rem(t, nt) * _TR
                _adj_copy(adj_ref, ring_ref, sem_ref, t, j).wait()

                @pl.when(t < nt)
                def _sweep1():
                    h = jnp.dot(ring_ref[j], s_ref[...],
                                preferred_element_type=jnp.float32)
                    h = jnp.maximum(h + b1_ref[...], 0.0)
                    g_ref[pl.ds(row, _TR), :] = jnp.dot(
                        h, w2_ref[...], preferred_element_type=jnp.float32)

                @pl.when(t >= nt)
                def _sweep2():
                    slot = jax.lax.rem(t - nt, 2)

                    @pl.when(t >= nt + 2)
                    def _reclaim():
                        _out_copy(stage_ref, out_ref, osem_ref, t - 2,
                                  slot).wait()

                    v = jnp.dot(ring_ref[j], g_ref[...],
                                preferred_element_type=jnp.float32)
                    v = v + b2_ref[...]
                    m = jnp.max(v, axis=1, keepdims=True)
                    lse = jnp.log(jnp.sum(jnp.exp(v - m), axis=1,
                                          keepdims=True)) + m
                    stage_ref[slot] = v - lse
                    _out_copy(stage_ref, out_ref, osem_ref, t, slot).start()

                @pl.when(t + _B < total)
                def _refill():
                    _adj_copy(adj_ref, ring_ref, sem_ref, t + _B, j).start()
            return 0

        jax.lax.fori_loop(0, total // _B, outer, 0)

        _out_copy(stage_ref, out_ref, osem_ref, total - 2, (nt - 2) % 2).wait()
        _out_copy(stage_ref, out_ref, osem_ref, total - 1, (nt - 1) % 2).wait()

    return _body


def kernel(x, adj, W1, b1, W2, b2):
    n, nfeat = x.shape
    nhid = W1.shape[1]
    nclass = W2.shape[1]
    b1r = b1.reshape(1, nhid)
    b2r = b2.reshape(1, nclass)

    hbm = pl.BlockSpec(memory_space=pltpu.MemorySpace.HBM)
    vmem = pl.BlockSpec(memory_space=pltpu.MemorySpace.VMEM)

    out = pl.pallas_call(
        _make_body(n, nfeat, nhid, nclass),
        in_specs=[hbm, hbm, vmem, vmem, vmem, vmem],
        out_specs=hbm,
        out_shape=jax.ShapeDtypeStruct((n, nclass), jnp.float32),
        scratch_shapes=[
            pltpu.VMEM((_B, _TR, n), jnp.float32),
            pltpu.VMEM((n, nfeat), jnp.float32),
            pltpu.VMEM((2, _TR, nclass), jnp.float32),
            pltpu.VMEM((n, nhid), jnp.float32),
            pltpu.VMEM((n, nclass), jnp.float32),
            pltpu.SemaphoreType.DMA((_B,)),
            pltpu.SemaphoreType.DMA,
            pltpu.SemaphoreType.DMA((2,)),
        ],
        compiler_params=pltpu.CompilerParams(
            vmem_limit_bytes=64 * 1024 * 1024),
    )(x, adj, W1, b1r, W2, b2r)

    return out
